# Initial kernel scaffold; baseline (speedup 1.0000x reference)
import jax
import jax.numpy as jnp
from jax.experimental import pallas as pl


def kernel(data, x, edge_index, W1, b1, W2, b2):
    raise NotImplementedError("write your pallas kernel here")


# speedup vs baseline: 7.2777x; 7.2777x over previous
import functools

import jax
import jax.numpy as jnp
from jax import lax
from jax.experimental import pallas as pl
from jax.experimental.pallas import tpu as pltpu
from jax.experimental.pallas import tpu_sc as plsc

N = 10000
E = 320000
D_IN = 128
D_HID = 256
D_OUT = 64
D2 = 128

NPAD = 10112
NPT = NPAD // 16
K = 128
CHUNKS_PAD = 2560
EPAD = CHUNKS_PAD * K
NB = 10

_SC_MESH = plsc.VectorSubcoreMesh(core_axis_name="c", subcore_axis_name="s")


@functools.partial(
    pl.kernel,
    out_type=jax.ShapeDtypeStruct((2, NPAD, 1), jnp.float32),
    mesh=_SC_MESH,
    scratch_types=[
        pltpu.VMEM_SHARED((NPAD, 1), jnp.float32),
        pltpu.VMEM((80, K), jnp.int32),
        pltpu.VMEM((K, 1), jnp.float32),
    ],
)
def _sc_degree(dst2d, init, deg_out, deg_sp, idx_v, ones_v):
    c = lax.axis_index("c")
    s = lax.axis_index("s")
    base = s * NPT
    pltpu.sync_copy(init.at[c, pl.ds(base, NPT)], deg_sp.at[pl.ds(base, NPT)])
    pltpu.sync_copy(init.at[0, pl.ds(0, K)], ones_v)
    row0 = (c * 16 + s) * 80
    pltpu.sync_copy(dst2d.at[pl.ds(row0, 80)], idx_v)
    plsc.subcore_barrier()

    def body(j, carry):
        pltpu.sync_copy(ones_v, deg_sp.at[idx_v.at[j]], add=True)
        return carry

    lax.fori_loop(0, 80, body, 0)
    plsc.subcore_barrier()
    pltpu.sync_copy(deg_sp.at[pl.ds(base, NPT)], deg_out.at[c, pl.ds(base, NPT)])


@functools.partial(
    pl.kernel,
    out_type=jax.ShapeDtypeStruct((2, NPAD, D_IN), jnp.float32),
    mesh=_SC_MESH,
    scratch_types=[
        pltpu.VMEM_SHARED((NPAD, D_IN), jnp.float32),
        pltpu.VMEM((8, K), jnp.int32),
        pltpu.VMEM((8, K), jnp.int32),
        pltpu.VMEM((K, D_IN), jnp.float32),
        pltpu.VMEM((16, D_IN), jnp.float32),
        pltpu.SemaphoreType.DMA,
    ],
)
def _sc_agg1(gflat, srcs2, dst2d, out, acc_sp, sidx, didx, rows, zbuf, sem):
    c = lax.axis_index("c")
    s = lax.axis_index("s")
    base = s * 624
    pltpu.sync_copy(gflat.at[pl.ds(c * N + base, 624)],
                    acc_sp.at[pl.ds(base, 624)])

    @pl.when(s == 15)
    def _():
        pltpu.sync_copy(gflat.at[pl.ds(c * N + 9984, 16)],
                        acc_sp.at[pl.ds(9984, 16)])

    @pl.when(s == 0)
    def _():
        for i in range(16):
            for k in range(D_IN // 16):
                zbuf[i, pl.ds(k * 16, 16)] = jnp.zeros((16,), jnp.float32)
        for t in range((NPAD - N) // 16):
            pltpu.sync_copy(zbuf, acc_sp.at[pl.ds(N + t * 16, 16)])

    row0 = s * 160
    plsc.subcore_barrier()

    def group(g, carry):
        goff = pl.multiple_of(row0 + g * 8, 8)
        pltpu.sync_copy(srcs2.at[c, pl.ds(goff, 8)], sidx)
        pltpu.sync_copy(dst2d.at[pl.ds(goff, 8)], didx)
        for j in range(8):
            pltpu.async_copy(gflat.at[sidx.at[j]], rows, sem).wait()
            pltpu.sync_copy(rows, acc_sp.at[didx.at[j]], add=True)
        return carry

    lax.fori_loop(0, 20, group, 0)
    plsc.subcore_barrier()
    base = s * NPT
    pltpu.sync_copy(acc_sp.at[pl.ds(base, NPT)], out.at[c, pl.ds(base, NPT)])


@functools.partial(
    pl.kernel,
    out_type=jax.ShapeDtypeStruct((2, NPAD, D2), jnp.float32),
    mesh=_SC_MESH,
    scratch_types=[
        pltpu.VMEM_SHARED((NPAD, D2), jnp.float32),
        pltpu.VMEM((80, K), jnp.int32),
        pltpu.VMEM((80, K), jnp.int32),
        pltpu.VMEM((K, D2), jnp.float32),
        pltpu.SemaphoreType.DMA,
    ],
)
def _sc_agg2(g2, g2init, src2d, dst2d, out, acc_sp, sidx, didx, rows, sem):
    c = lax.axis_index("c")
    s = lax.axis_index("s")
    base = s * NPT
    pltpu.sync_copy(g2init.at[c, pl.ds(base, NPT)], acc_sp.at[pl.ds(base, NPT)])
    row0 = (c * 16 + s) * 80
    pltpu.sync_copy(src2d.at[pl.ds(row0, 80)], sidx)
    pltpu.sync_copy(dst2d.at[pl.ds(row0, 80)], didx)
    plsc.subcore_barrier()

    def body(j, carry):
        pltpu.async_copy(g2.at[sidx.at[j]], rows, sem).wait()
        pltpu.sync_copy(rows, acc_sp.at[didx.at[j]], add=True)
        return carry

    lax.fori_loop(0, 80, body, 0)
    plsc.subcore_barrier()
    pltpu.sync_copy(acc_sp.at[pl.ds(base, NPT)], out.at[c, pl.ds(base, NPT)])


def _tc_scale_matmul_body(x_ref, w_ref, degs_ref, o_ref):
    deg = degs_ref[0] + degs_ref[1]
    dis = lax.rsqrt(deg)
    g = jnp.dot(x_ref[...], w_ref[...], preferred_element_type=jnp.float32)
    g = g * dis
    o_ref[0] = g[:, :D_IN]
    o_ref[1] = g[:, D_IN:]


def _tc_layer1_post_body(a_ref, degs_ref, b1_ref, w2_ref, o_ref):
    deg = degs_ref[0] + degs_ref[1]
    dis = lax.rsqrt(deg)
    h = jnp.concatenate([a_ref[0], a_ref[1]], axis=1) * dis + b1_ref[...]
    h = jnp.maximum(h, 0.0)
    g2 = jnp.dot(h, w2_ref[...], preferred_element_type=jnp.float32)
    o_ref[...] = jnp.concatenate(
        [g2 * dis, jnp.zeros((_BR, D2 - D_OUT), jnp.float32)], axis=1)


def _tc_final_body(a_ref, degs_ref, b2_ref, o_ref):
    deg = degs_ref[0] + degs_ref[1]
    dis = lax.rsqrt(deg)
    o_ref[...] = (a_ref[0, :, :D_OUT] + a_ref[1, :, :D_OUT]) * dis + b2_ref[...]


_BR = N // NB


def _deg_spec():
    return pl.BlockSpec((2, _BR, 1), lambda n: (0, n, 0))


_tc_scale_matmul = pl.pallas_call(
    _tc_scale_matmul_body,
    grid=(NB,),
    in_specs=[
        pl.BlockSpec((_BR, D_IN), lambda n: (n, 0)),
        pl.BlockSpec((D_IN, D_HID), lambda n: (0, 0)),
        _deg_spec(),
    ],
    out_specs=pl.BlockSpec((2, _BR, D_IN), lambda n: (0, n, 0)),
    out_shape=jax.ShapeDtypeStruct((2, N, D_IN), jnp.float32),
)

_tc_layer1_post = pl.pallas_call(
    _tc_layer1_post_body,
    grid=(NB,),
    in_specs=[
        pl.BlockSpec((2, _BR, D_IN), lambda n: (0, n, 0)),
        _deg_spec(),
        pl.BlockSpec((1, D_HID), lambda n: (0, 0)),
        pl.BlockSpec((D_HID, D_OUT), lambda n: (0, 0)),
    ],
    out_specs=pl.BlockSpec((_BR, D2), lambda n: (n, 0)),
    out_shape=jax.ShapeDtypeStruct((N, D2), jnp.float32),
)

_tc_final = pl.pallas_call(
    _tc_final_body,
    grid=(NB,),
    in_specs=[
        pl.BlockSpec((2, _BR, D2), lambda n: (0, n, 0)),
        _deg_spec(),
        pl.BlockSpec((1, D_OUT), lambda n: (0, 0)),
    ],
    out_specs=pl.BlockSpec((_BR, D_OUT), lambda n: (n, 0)),
    out_shape=jax.ShapeDtypeStruct((N, D_OUT), jnp.float32),
)


def kernel(data, x, edge_index, W1, b1, W2, b2):
    src = edge_index[0].astype(jnp.int32)
    dst = edge_index[1].astype(jnp.int32)
    pad = EPAD - E
    src_p = jnp.concatenate([src, jnp.zeros((pad,), jnp.int32)])
    dst_p = jnp.concatenate([dst, jnp.full((pad,), N, jnp.int32)])
    src2d = src_p.reshape(CHUNKS_PAD, K)
    dst2d = dst_p.reshape(CHUNKS_PAD, K)
    srcs2 = jnp.stack([src2d, src2d + N])

    deg_init = jnp.zeros((2, NPAD, 1), jnp.float32).at[0, :N, 0].set(1.0)
    degs = _sc_degree(dst2d, deg_init)

    g1 = _tc_scale_matmul(x, W1, degs)
    agg1 = _sc_agg1(g1.reshape(2 * N, D_IN), srcs2, dst2d)

    g2 = _tc_layer1_post(agg1, degs, b1.reshape(1, D_HID), W2)
    g2init = jnp.stack([
        jnp.pad(g2, ((0, NPAD - N), (0, 0))),
        jnp.zeros((NPAD, D2), jnp.float32),
    ])
    agg2 = _sc_agg2(g2, g2init, src2d, dst2d)

    return _tc_final(agg2, degs, b2.reshape(1, D_OUT))


# speedup vs baseline: 8.1186x; 1.1155x over previous
import functools

import jax
import jax.numpy as jnp
from jax import lax
from jax.experimental import pallas as pl
from jax.experimental.pallas import tpu as pltpu
from jax.experimental.pallas import tpu_sc as plsc

N = 10000
E = 320000
D_IN = 128
D_HID = 256
D_OUT = 64
D2 = 128

NPAD = 10112
NPT = NPAD // 16
K = 128
CHUNKS_PAD = 2560
EPAD = CHUNKS_PAD * K
NB = 10
GRP = 16

_SC_MESH = plsc.VectorSubcoreMesh(core_axis_name="c", subcore_axis_name="s")


@functools.partial(
    pl.kernel,
    out_type=jax.ShapeDtypeStruct((2, NPAD, 1), jnp.float32),
    mesh=_SC_MESH,
    scratch_types=[
        pltpu.VMEM_SHARED((NPAD, 1), jnp.float32),
        pltpu.VMEM((80, K), jnp.int32),
        pltpu.VMEM((K, 1), jnp.float32),
    ],
)
def _sc_degree(dst2d, init, deg_out, deg_sp, idx_v, ones_v):
    c = lax.axis_index("c")
    s = lax.axis_index("s")
    base = s * NPT
    pltpu.sync_copy(init.at[c, pl.ds(base, NPT)], deg_sp.at[pl.ds(base, NPT)])
    pltpu.sync_copy(init.at[0, pl.ds(0, K)], ones_v)
    row0 = (c * 16 + s) * 80
    pltpu.sync_copy(dst2d.at[pl.ds(row0, 80)], idx_v)
    plsc.subcore_barrier()

    def body(j, carry):
        pltpu.sync_copy(ones_v, deg_sp.at[idx_v.at[j]], add=True)
        return carry

    lax.fori_loop(0, 80, body, 0)
    plsc.subcore_barrier()
    pltpu.sync_copy(deg_sp.at[pl.ds(base, NPT)], deg_out.at[c, pl.ds(base, NPT)])


@functools.partial(
    pl.kernel,
    out_type=jax.ShapeDtypeStruct((2, NPAD, D_IN), jnp.float32),
    mesh=_SC_MESH,
    scratch_types=[
        pltpu.VMEM_SHARED((NPAD, D_IN), jnp.float32),
        pltpu.VMEM((2, GRP, K), jnp.int32),
        pltpu.VMEM((2, GRP, K), jnp.int32),
        pltpu.VMEM((2, K, D_IN), jnp.float32),
        pltpu.VMEM((16, D_IN), jnp.float32),
        pltpu.SemaphoreType.DMA,
    ],
)
def _sc_agg1(gflat, srcs2, dst2d, out, acc_sp, sidx, didx, rows, zbuf, sem):
    c = lax.axis_index("c")
    s = lax.axis_index("s")
    base = s * 624
    pltpu.sync_copy(gflat.at[pl.ds(c * N + base, 624)],
                    acc_sp.at[pl.ds(base, 624)])

    @pl.when(s == 15)
    def _():
        pltpu.sync_copy(gflat.at[pl.ds(c * N + 9984, 16)],
                        acc_sp.at[pl.ds(9984, 16)])

    @pl.when(s == 0)
    def _():
        for i in range(16):
            for k in range(D_IN // 16):
                zbuf[i, pl.ds(k * 16, 16)] = jnp.zeros((16,), jnp.float32)
        for t in range((NPAD - N) // 16):
            pltpu.sync_copy(zbuf, acc_sp.at[pl.ds(N + t * 16, 16)])

    row0 = s * 160
    nch = 160
    pltpu.sync_copy(srcs2.at[c, pl.ds(pl.multiple_of(row0, 8), GRP)],
                    sidx.at[0])
    pltpu.sync_copy(dst2d.at[pl.ds(pl.multiple_of(row0, 8), GRP)],
                    didx.at[0])
    pltpu.async_copy(gflat.at[sidx.at[0, 0]], rows.at[0], sem)
    plsc.subcore_barrier()

    def body(j, carry):
        g = j // GRP

        @pl.when((j % GRP == 0) & (j + GRP < nch))
        def _():
            goff = pl.multiple_of(row0 + (g + 1) * GRP, 8)
            pltpu.sync_copy(srcs2.at[c, pl.ds(goff, GRP)],
                            sidx.at[(g + 1) % 2])
            pltpu.sync_copy(dst2d.at[pl.ds(goff, GRP)],
                            didx.at[(g + 1) % 2])

        pltpu.make_async_copy(gflat.at[pl.ds(0, K)], rows.at[j % 2],
                              sem).wait()

        @pl.when(j + 1 < nch)
        def _():
            jn = j + 1
            pltpu.async_copy(gflat.at[sidx.at[(jn // GRP) % 2, jn % GRP]],
                             rows.at[jn % 2], sem)

        pltpu.sync_copy(rows.at[j % 2],
                        acc_sp.at[didx.at[g % 2, j % GRP]], add=True)
        return carry

    lax.fori_loop(0, nch, body, 0)
    plsc.subcore_barrier()
    base = s * NPT
    pltpu.sync_copy(acc_sp.at[pl.ds(base, NPT)], out.at[c, pl.ds(base, NPT)])


@functools.partial(
    pl.kernel,
    out_type=jax.ShapeDtypeStruct((2, NPAD, D2), jnp.float32),
    mesh=_SC_MESH,
    scratch_types=[
        pltpu.VMEM_SHARED((NPAD, D2), jnp.float32),
        pltpu.VMEM((2, GRP, K), jnp.int32),
        pltpu.VMEM((2, GRP, K), jnp.int32),
        pltpu.VMEM((2, K, D2), jnp.float32),
        pltpu.SemaphoreType.DMA,
    ],
)
def _sc_agg2(g2, g2init, src2d, dst2d, out, acc_sp, sidx, didx, rows, sem):
    c = lax.axis_index("c")
    s = lax.axis_index("s")
    base = s * NPT
    pltpu.sync_copy(g2init.at[c, pl.ds(base, NPT)], acc_sp.at[pl.ds(base, NPT)])
    row0 = (c * 16 + s) * 80
    nch = 80
    pltpu.sync_copy(src2d.at[pl.ds(pl.multiple_of(row0, 8), GRP)], sidx.at[0])
    pltpu.sync_copy(dst2d.at[pl.ds(pl.multiple_of(row0, 8), GRP)], didx.at[0])
    pltpu.async_copy(g2.at[sidx.at[0, 0]], rows.at[0], sem)
    plsc.subcore_barrier()

    def body(j, carry):
        g = j // GRP

        @pl.when((j % GRP == 0) & (j + GRP < nch))
        def _():
            goff = pl.multiple_of(row0 + (g + 1) * GRP, 8)
            pltpu.sync_copy(src2d.at[pl.ds(goff, GRP)], sidx.at[(g + 1) % 2])
            pltpu.sync_copy(dst2d.at[pl.ds(goff, GRP)], didx.at[(g + 1) % 2])

        pltpu.make_async_copy(g2.at[pl.ds(0, K)], rows.at[j % 2], sem).wait()

        @pl.when(j + 1 < nch)
        def _():
            jn = j + 1
            pltpu.async_copy(g2.at[sidx.at[(jn // GRP) % 2, jn % GRP]],
                             rows.at[jn % 2], sem)

        pltpu.sync_copy(rows.at[j % 2],
                        acc_sp.at[didx.at[g % 2, j % GRP]], add=True)
        return carry

    lax.fori_loop(0, nch, body, 0)
    plsc.subcore_barrier()
    pltpu.sync_copy(acc_sp.at[pl.ds(base, NPT)], out.at[c, pl.ds(base, NPT)])


def _tc_scale_matmul_body(x_ref, w_ref, degs_ref, o_ref):
    deg = degs_ref[0] + degs_ref[1]
    dis = lax.rsqrt(deg)
    g = jnp.dot(x_ref[...], w_ref[...], preferred_element_type=jnp.float32)
    g = g * dis
    o_ref[0] = g[:, :D_IN]
    o_ref[1] = g[:, D_IN:]


def _tc_layer1_post_body(a_ref, degs_ref, b1_ref, w2_ref, o_ref):
    deg = degs_ref[0] + degs_ref[1]
    dis = lax.rsqrt(deg)
    h = jnp.concatenate([a_ref[0], a_ref[1]], axis=1) * dis + b1_ref[...]
    h = jnp.maximum(h, 0.0)
    g2 = jnp.dot(h, w2_ref[...], preferred_element_type=jnp.float32)
    o_ref[...] = jnp.concatenate(
        [g2 * dis, jnp.zeros((_BR, D2 - D_OUT), jnp.float32)], axis=1)


def _tc_final_body(a_ref, degs_ref, b2_ref, o_ref):
    deg = degs_ref[0] + degs_ref[1]
    dis = lax.rsqrt(deg)
    o_ref[...] = (a_ref[0, :, :D_OUT] + a_ref[1, :, :D_OUT]) * dis + b2_ref[...]


_BR = N // NB


def _deg_spec():
    return pl.BlockSpec((2, _BR, 1), lambda n: (0, n, 0))


_tc_scale_matmul = pl.pallas_call(
    _tc_scale_matmul_body,
    grid=(NB,),
    in_specs=[
        pl.BlockSpec((_BR, D_IN), lambda n: (n, 0)),
        pl.BlockSpec((D_IN, D_HID), lambda n: (0, 0)),
        _deg_spec(),
    ],
    out_specs=pl.BlockSpec((2, _BR, D_IN), lambda n: (0, n, 0)),
    out_shape=jax.ShapeDtypeStruct((2, N, D_IN), jnp.float32),
)

_tc_layer1_post = pl.pallas_call(
    _tc_layer1_post_body,
    grid=(NB,),
    in_specs=[
        pl.BlockSpec((2, _BR, D_IN), lambda n: (0, n, 0)),
        _deg_spec(),
        pl.BlockSpec((1, D_HID), lambda n: (0, 0)),
        pl.BlockSpec((D_HID, D_OUT), lambda n: (0, 0)),
    ],
    out_specs=pl.BlockSpec((_BR, D2), lambda n: (n, 0)),
    out_shape=jax.ShapeDtypeStruct((N, D2), jnp.float32),
)

_tc_final = pl.pallas_call(
    _tc_final_body,
    grid=(NB,),
    in_specs=[
        pl.BlockSpec((2, _BR, D2), lambda n: (0, n, 0)),
        _deg_spec(),
        pl.BlockSpec((1, D_OUT), lambda n: (0, 0)),
    ],
    out_specs=pl.BlockSpec((_BR, D_OUT), lambda n: (n, 0)),
    out_shape=jax.ShapeDtypeStruct((N, D_OUT), jnp.float32),
)


def kernel(data, x, edge_index, W1, b1, W2, b2):
    src = edge_index[0].astype(jnp.int32)
    dst = edge_index[1].astype(jnp.int32)
    pad = EPAD - E
    src_p = jnp.concatenate([src, jnp.zeros((pad,), jnp.int32)])
    dst_p = jnp.concatenate([dst, jnp.full((pad,), N, jnp.int32)])
    src2d = src_p.reshape(CHUNKS_PAD, K)
    dst2d = dst_p.reshape(CHUNKS_PAD, K)
    srcs2 = jnp.stack([src2d, src2d + N])

    deg_init = jnp.zeros((2, NPAD, 1), jnp.float32).at[0, :N, 0].set(1.0)
    degs = _sc_degree(dst2d, deg_init)

    g1 = _tc_scale_matmul(x, W1, degs)
    agg1 = _sc_agg1(g1.reshape(2 * N, D_IN), srcs2, dst2d)

    g2 = _tc_layer1_post(agg1, degs, b1.reshape(1, D_HID), W2)
    g2init = jnp.stack([
        jnp.pad(g2, ((0, NPAD - N), (0, 0))),
        jnp.zeros((NPAD, D2), jnp.float32),
    ])
    agg2 = _sc_agg2(g2, g2init, src2d, dst2d)

    return _tc_final(agg2, degs, b2.reshape(1, D_OUT))


# speedup vs baseline: 8.4948x; 1.0463x over previous
import functools

import jax
import jax.numpy as jnp
from jax import lax
from jax.experimental import pallas as pl
from jax.experimental.pallas import tpu as pltpu
from jax.experimental.pallas import tpu_sc as plsc

N = 10000
E = 320000
D_IN = 128
D_HID = 256
D_OUT = 64
D2 = 128

NPAD = 10112
NPT = NPAD // 16
K = 128
CHUNKS_PAD = 2560
EPAD = CHUNKS_PAD * K
NB = 10
GRP = 16

_SC_MESH = plsc.VectorSubcoreMesh(core_axis_name="c", subcore_axis_name="s")


@functools.partial(
    pl.kernel,
    out_type=jax.ShapeDtypeStruct((2, NPAD, 1), jnp.float32),
    mesh=_SC_MESH,
    scratch_types=[
        pltpu.VMEM_SHARED((NPAD, 1), jnp.float32),
        pltpu.VMEM((80, K), jnp.int32),
        pltpu.VMEM((K, 1), jnp.float32),
    ],
)
def _sc_degree(dst2d, init, deg_out, deg_sp, idx_v, ones_v):
    c = lax.axis_index("c")
    s = lax.axis_index("s")
    base = s * NPT
    pltpu.sync_copy(init.at[c, pl.ds(base, NPT)], deg_sp.at[pl.ds(base, NPT)])
    pltpu.sync_copy(init.at[0, pl.ds(0, K)], ones_v)
    row0 = (c * 16 + s) * 80
    pltpu.sync_copy(dst2d.at[pl.ds(row0, 80)], idx_v)
    plsc.subcore_barrier()

    def body(j, carry):
        pltpu.sync_copy(ones_v, deg_sp.at[idx_v.at[j]], add=True)
        return carry

    lax.fori_loop(0, 80, body, 0)
    plsc.subcore_barrier()
    pltpu.sync_copy(deg_sp.at[pl.ds(base, NPT)], deg_out.at[c, pl.ds(base, NPT)])


@functools.partial(
    pl.kernel,
    out_type=jax.ShapeDtypeStruct((2, NPAD, D_IN), jnp.float32),
    mesh=_SC_MESH,
    scratch_types=[
        pltpu.VMEM_SHARED((NPAD, D_IN), jnp.float32),
        pltpu.VMEM((2, GRP, K), jnp.int32),
        pltpu.VMEM((2, GRP, K), jnp.int32),
        pltpu.VMEM((2, K, D_IN), jnp.float32),
        pltpu.VMEM((16, D_IN), jnp.float32),
        pltpu.SemaphoreType.DMA((2,)),
        pltpu.SemaphoreType.DMA((2,)),
    ],
)
def _sc_agg1(gflat, srcs2, dst2d, out, acc_sp, sidx, didx, rows, zbuf, semg,
             sems):
    c = lax.axis_index("c")
    s = lax.axis_index("s")
    base = s * 624
    pltpu.sync_copy(gflat.at[pl.ds(c * N + base, 624)],
                    acc_sp.at[pl.ds(base, 624)])

    @pl.when(s == 15)
    def _():
        pltpu.sync_copy(gflat.at[pl.ds(c * N + 9984, 16)],
                        acc_sp.at[pl.ds(9984, 16)])

    @pl.when(s == 0)
    def _():
        for i in range(16):
            for k in range(D_IN // 16):
                zbuf[i, pl.ds(k * 16, 16)] = jnp.zeros((16,), jnp.float32)
        for t in range((NPAD - N) // 16):
            pltpu.sync_copy(zbuf, acc_sp.at[pl.ds(N + t * 16, 16)])

    row0 = s * 160
    nch = 160
    pltpu.sync_copy(srcs2.at[c, pl.ds(pl.multiple_of(row0, 8), GRP)],
                    sidx.at[0])
    pltpu.sync_copy(dst2d.at[pl.ds(pl.multiple_of(row0, 8), GRP)],
                    didx.at[0])
    pltpu.async_copy(gflat.at[sidx.at[0, 0]], rows.at[0], semg.at[0])
    plsc.subcore_barrier()

    def body(j, carry):
        g = j // GRP
        b = j % 2
        bn = (j + 1) % 2

        @pl.when((j % GRP == 0) & (j + GRP < nch))
        def _():
            goff = pl.multiple_of(row0 + (g + 1) * GRP, 8)
            pltpu.sync_copy(srcs2.at[c, pl.ds(goff, GRP)],
                            sidx.at[(g + 1) % 2])
            pltpu.sync_copy(dst2d.at[pl.ds(goff, GRP)],
                            didx.at[(g + 1) % 2])

        @pl.when(j >= 1)
        def _():
            pltpu.make_async_copy(gflat.at[pl.ds(0, K)], rows.at[bn],
                                  sems.at[bn]).wait()

        @pl.when(j + 1 < nch)
        def _():
            jn = j + 1
            pltpu.async_copy(gflat.at[sidx.at[(jn // GRP) % 2, jn % GRP]],
                             rows.at[bn], semg.at[bn])

        pltpu.make_async_copy(gflat.at[pl.ds(0, K)], rows.at[b],
                              semg.at[b]).wait()
        pltpu.async_copy(rows.at[b], acc_sp.at[didx.at[g % 2, j % GRP]],
                         sems.at[b], add=True)
        return carry

    lax.fori_loop(0, nch, body, 0)
    pltpu.make_async_copy(gflat.at[pl.ds(0, K)], rows.at[(nch - 1) % 2],
                          sems.at[(nch - 1) % 2]).wait()
    plsc.subcore_barrier()
    base = s * NPT
    pltpu.sync_copy(acc_sp.at[pl.ds(base, NPT)], out.at[c, pl.ds(base, NPT)])


@functools.partial(
    pl.kernel,
    out_type=jax.ShapeDtypeStruct((2, NPAD, D2), jnp.float32),
    mesh=_SC_MESH,
    scratch_types=[
        pltpu.VMEM_SHARED((NPAD, D2), jnp.float32),
        pltpu.VMEM((2, GRP, K), jnp.int32),
        pltpu.VMEM((2, GRP, K), jnp.int32),
        pltpu.VMEM((2, K, D2), jnp.float32),
        pltpu.SemaphoreType.DMA((2,)),
        pltpu.SemaphoreType.DMA((2,)),
    ],
)
def _sc_agg2(g2, g2init, src2d, dst2d, out, acc_sp, sidx, didx, rows, semg,
             sems):
    c = lax.axis_index("c")
    s = lax.axis_index("s")
    base = s * NPT
    pltpu.sync_copy(g2init.at[c, pl.ds(base, NPT)], acc_sp.at[pl.ds(base, NPT)])
    row0 = (c * 16 + s) * 80
    nch = 80
    pltpu.sync_copy(src2d.at[pl.ds(pl.multiple_of(row0, 8), GRP)], sidx.at[0])
    pltpu.sync_copy(dst2d.at[pl.ds(pl.multiple_of(row0, 8), GRP)], didx.at[0])
    pltpu.async_copy(g2.at[sidx.at[0, 0]], rows.at[0], semg.at[0])
    plsc.subcore_barrier()

    def body(j, carry):
        g = j // GRP
        b = j % 2
        bn = (j + 1) % 2

        @pl.when((j % GRP == 0) & (j + GRP < nch))
        def _():
            goff = pl.multiple_of(row0 + (g + 1) * GRP, 8)
            pltpu.sync_copy(src2d.at[pl.ds(goff, GRP)], sidx.at[(g + 1) % 2])
            pltpu.sync_copy(dst2d.at[pl.ds(goff, GRP)], didx.at[(g + 1) % 2])

        @pl.when(j >= 1)
        def _():
            pltpu.make_async_copy(g2.at[pl.ds(0, K)], rows.at[bn],
                                  sems.at[bn]).wait()

        @pl.when(j + 1 < nch)
        def _():
            jn = j + 1
            pltpu.async_copy(g2.at[sidx.at[(jn // GRP) % 2, jn % GRP]],
                             rows.at[bn], semg.at[bn])

        pltpu.make_async_copy(g2.at[pl.ds(0, K)], rows.at[b],
                              semg.at[b]).wait()
        pltpu.async_copy(rows.at[b], acc_sp.at[didx.at[g % 2, j % GRP]],
                         sems.at[b], add=True)
        return carry

    lax.fori_loop(0, nch, body, 0)
    pltpu.make_async_copy(g2.at[pl.ds(0, K)], rows.at[(nch - 1) % 2],
                          sems.at[(nch - 1) % 2]).wait()
    plsc.subcore_barrier()
    pltpu.sync_copy(acc_sp.at[pl.ds(base, NPT)], out.at[c, pl.ds(base, NPT)])


def _tc_scale_matmul_body(x_ref, w_ref, degs_ref, o_ref):
    deg = degs_ref[0] + degs_ref[1]
    dis = lax.rsqrt(deg)
    g = jnp.dot(x_ref[...], w_ref[...], preferred_element_type=jnp.float32)
    g = g * dis
    o_ref[0] = g[:, :D_IN]
    o_ref[1] = g[:, D_IN:]


def _tc_layer1_post_body(a_ref, degs_ref, b1_ref, w2_ref, o_ref):
    deg = degs_ref[0] + degs_ref[1]
    dis = lax.rsqrt(deg)
    h = jnp.concatenate([a_ref[0], a_ref[1]], axis=1) * dis + b1_ref[...]
    h = jnp.maximum(h, 0.0)
    g2 = jnp.dot(h, w2_ref[...], preferred_element_type=jnp.float32)
    o_ref[...] = jnp.concatenate(
        [g2 * dis, jnp.zeros((_BR, D2 - D_OUT), jnp.float32)], axis=1)


def _tc_final_body(a_ref, degs_ref, b2_ref, o_ref):
    deg = degs_ref[0] + degs_ref[1]
    dis = lax.rsqrt(deg)
    o_ref[...] = (a_ref[0, :, :D_OUT] + a_ref[1, :, :D_OUT]) * dis + b2_ref[...]


_BR = N // NB


def _deg_spec():
    return pl.BlockSpec((2, _BR, 1), lambda n: (0, n, 0))


_tc_scale_matmul = pl.pallas_call(
    _tc_scale_matmul_body,
    grid=(NB,),
    in_specs=[
        pl.BlockSpec((_BR, D_IN), lambda n: (n, 0)),
        pl.BlockSpec((D_IN, D_HID), lambda n: (0, 0)),
        _deg_spec(),
    ],
    out_specs=pl.BlockSpec((2, _BR, D_IN), lambda n: (0, n, 0)),
    out_shape=jax.ShapeDtypeStruct((2, N, D_IN), jnp.float32),
)

_tc_layer1_post = pl.pallas_call(
    _tc_layer1_post_body,
    grid=(NB,),
    in_specs=[
        pl.BlockSpec((2, _BR, D_IN), lambda n: (0, n, 0)),
        _deg_spec(),
        pl.BlockSpec((1, D_HID), lambda n: (0, 0)),
        pl.BlockSpec((D_HID, D_OUT), lambda n: (0, 0)),
    ],
    out_specs=pl.BlockSpec((_BR, D2), lambda n: (n, 0)),
    out_shape=jax.ShapeDtypeStruct((N, D2), jnp.float32),
)

_tc_final = pl.pallas_call(
    _tc_final_body,
    grid=(NB,),
    in_specs=[
        pl.BlockSpec((2, _BR, D2), lambda n: (0, n, 0)),
        _deg_spec(),
        pl.BlockSpec((1, D_OUT), lambda n: (0, 0)),
    ],
    out_specs=pl.BlockSpec((_BR, D_OUT), lambda n: (n, 0)),
    out_shape=jax.ShapeDtypeStruct((N, D_OUT), jnp.float32),
)


def kernel(data, x, edge_index, W1, b1, W2, b2):
    src = edge_index[0].astype(jnp.int32)
    dst = edge_index[1].astype(jnp.int32)
    pad = EPAD - E
    src_p = jnp.concatenate([src, jnp.zeros((pad,), jnp.int32)])
    dst_p = jnp.concatenate([dst, jnp.full((pad,), N, jnp.int32)])
    src2d = src_p.reshape(CHUNKS_PAD, K)
    dst2d = dst_p.reshape(CHUNKS_PAD, K)
    srcs2 = jnp.stack([src2d, src2d + N])

    deg_init = jnp.zeros((2, NPAD, 1), jnp.float32).at[0, :N, 0].set(1.0)
    degs = _sc_degree(dst2d, deg_init)

    g1 = _tc_scale_matmul(x, W1, degs)
    agg1 = _sc_agg1(g1.reshape(2 * N, D_IN), srcs2, dst2d)

    g2 = _tc_layer1_post(agg1, degs, b1.reshape(1, D_HID), W2)
    g2init = jnp.stack([
        jnp.pad(g2, ((0, NPAD - N), (0, 0))),
        jnp.zeros((NPAD, D2), jnp.float32),
    ])
    agg2 = _sc_agg2(g2, g2init, src2d, dst2d)

    return _tc_final(agg2, degs, b2.reshape(1, D_OUT))


# speedup vs baseline: 25.9907x; 3.0596x over previous
import functools

import jax
import jax.numpy as jnp
from jax import lax
from jax.experimental import pallas as pl
from jax.experimental.pallas import tpu as pltpu
from jax.experimental.pallas import tpu_sc as plsc

N = 10000
E = 320000
D_IN = 128
D_HID = 256
D_OUT = 64
D2 = 128

NPAD = 10112
NPT = NPAD // 16
K = 128
CHUNKS_PAD = 2560
EPAD = CHUNKS_PAD * K
NB = 10
GRP = 16

_SC_MESH = plsc.VectorSubcoreMesh(core_axis_name="c", subcore_axis_name="s")


@functools.partial(
    pl.kernel,
    out_type=jax.ShapeDtypeStruct((2, NPAD, 1), jnp.float32),
    mesh=_SC_MESH,
    scratch_types=[
        pltpu.VMEM_SHARED((NPAD, 1), jnp.float32),
        pltpu.VMEM((80, K), jnp.int32),
        pltpu.VMEM((K, 1), jnp.float32),
    ],
)
def _sc_degree(dst2d, init, deg_out, deg_sp, idx_v, ones_v):
    c = lax.axis_index("c")
    s = lax.axis_index("s")
    base = s * NPT
    pltpu.sync_copy(init.at[c, pl.ds(base, NPT)], deg_sp.at[pl.ds(base, NPT)])
    pltpu.sync_copy(init.at[0, pl.ds(0, K)], ones_v)
    row0 = (c * 16 + s) * 80
    pltpu.sync_copy(dst2d.at[pl.ds(row0, 80)], idx_v)
    plsc.subcore_barrier()

    def body(j, carry):
        pltpu.sync_copy(ones_v, deg_sp.at[idx_v.at[j]], add=True)
        return carry

    lax.fori_loop(0, 80, body, 0)
    plsc.subcore_barrier()
    pltpu.sync_copy(deg_sp.at[pl.ds(base, NPT)], deg_out.at[c, pl.ds(base, NPT)])


@functools.partial(
    pl.kernel,
    out_type=jax.ShapeDtypeStruct((2, NPAD, D_IN), jnp.float32),
    mesh=_SC_MESH,
    scratch_types=[
        pltpu.VMEM_SHARED((NPAD, D_IN), jnp.float32),
        pltpu.VMEM((2, GRP, K), jnp.int32),
        pltpu.VMEM((2, GRP, K), jnp.int32),
        pltpu.VMEM((2, K, D_IN), jnp.float32),
        pltpu.VMEM((16, D_IN), jnp.float32),
        pltpu.SemaphoreType.DMA((2,)),
        pltpu.SemaphoreType.DMA((2,)),
    ],
)
def _sc_agg1(gflat, srcs2, dst2d, out, acc_sp, sidx, didx, rows, zbuf, semg,
             sems):
    c = lax.axis_index("c")
    s = lax.axis_index("s")
    base = s * 624
    pltpu.sync_copy(gflat.at[pl.ds(c * N + base, 624)],
                    acc_sp.at[pl.ds(base, 624)])

    @pl.when(s == 15)
    def _():
        pltpu.sync_copy(gflat.at[pl.ds(c * N + 9984, 16)],
                        acc_sp.at[pl.ds(9984, 16)])

    @pl.when(s == 0)
    def _():
        for i in range(16):
            for k in range(D_IN // 16):
                zbuf[i, pl.ds(k * 16, 16)] = jnp.zeros((16,), jnp.float32)
        for t in range((NPAD - N) // 16):
            pltpu.sync_copy(zbuf, acc_sp.at[pl.ds(N + t * 16, 16)])

    row0 = s * 160
    nch = 160
    pltpu.sync_copy(srcs2.at[c, pl.ds(pl.multiple_of(row0, 8), GRP)],
                    sidx.at[0])
    pltpu.sync_copy(dst2d.at[pl.ds(pl.multiple_of(row0, 8), GRP)],
                    didx.at[0])
    pltpu.async_copy(gflat.at[sidx.at[0, 0]], rows.at[0], semg.at[0])
    plsc.subcore_barrier()

    def body(j, carry):
        g = j // GRP
        b = j % 2
        bn = (j + 1) % 2

        @pl.when((j % GRP == 0) & (j + GRP < nch))
        def _():
            goff = pl.multiple_of(row0 + (g + 1) * GRP, 8)
            pltpu.sync_copy(srcs2.at[c, pl.ds(goff, GRP)],
                            sidx.at[(g + 1) % 2])
            pltpu.sync_copy(dst2d.at[pl.ds(goff, GRP)],
                            didx.at[(g + 1) % 2])

        @pl.when(j >= 1)
        def _():
            pltpu.make_async_copy(gflat.at[pl.ds(0, K)], rows.at[bn],
                                  sems.at[bn]).wait()

        @pl.when(j + 1 < nch)
        def _():
            jn = j + 1
            pltpu.async_copy(gflat.at[sidx.at[(jn // GRP) % 2, jn % GRP]],
                             rows.at[bn], semg.at[bn])

        pltpu.make_async_copy(gflat.at[pl.ds(0, K)], rows.at[b],
                              semg.at[b]).wait()
        pltpu.async_copy(rows.at[b], acc_sp.at[didx.at[g % 2, j % GRP]],
                         sems.at[b], add=True)
        return carry

    lax.fori_loop(0, nch, body, 0)
    pltpu.make_async_copy(gflat.at[pl.ds(0, K)], rows.at[(nch - 1) % 2],
                          sems.at[(nch - 1) % 2]).wait()
    plsc.subcore_barrier()
    base = s * NPT
    pltpu.sync_copy(acc_sp.at[pl.ds(base, NPT)], out.at[c, pl.ds(base, NPT)])


@functools.partial(
    pl.kernel,
    out_type=jax.ShapeDtypeStruct((2, NPAD, D2), jnp.float32),
    mesh=_SC_MESH,
    scratch_types=[
        pltpu.VMEM_SHARED((NPAD, D2), jnp.float32),
        pltpu.VMEM((2, GRP, K), jnp.int32),
        pltpu.VMEM((2, GRP, K), jnp.int32),
        pltpu.VMEM((2, K, D2), jnp.float32),
        pltpu.SemaphoreType.DMA((2,)),
        pltpu.SemaphoreType.DMA((2,)),
    ],
)
def _sc_agg2(g2, g2init, src2d, dst2d, out, acc_sp, sidx, didx, rows, semg,
             sems):
    c = lax.axis_index("c")
    s = lax.axis_index("s")
    base = s * NPT
    pltpu.sync_copy(g2init.at[c, pl.ds(base, NPT)], acc_sp.at[pl.ds(base, NPT)])
    row0 = (c * 16 + s) * 80
    nch = 80
    pltpu.sync_copy(src2d.at[pl.ds(pl.multiple_of(row0, 8), GRP)], sidx.at[0])
    pltpu.sync_copy(dst2d.at[pl.ds(pl.multiple_of(row0, 8), GRP)], didx.at[0])
    pltpu.async_copy(g2.at[sidx.at[0, 0]], rows.at[0], semg.at[0])
    plsc.subcore_barrier()

    def body(j, carry):
        g = j // GRP
        b = j % 2
        bn = (j + 1) % 2

        @pl.when((j % GRP == 0) & (j + GRP < nch))
        def _():
            goff = pl.multiple_of(row0 + (g + 1) * GRP, 8)
            pltpu.sync_copy(src2d.at[pl.ds(goff, GRP)], sidx.at[(g + 1) % 2])
            pltpu.sync_copy(dst2d.at[pl.ds(goff, GRP)], didx.at[(g + 1) % 2])

        @pl.when(j >= 1)
        def _():
            pltpu.make_async_copy(g2.at[pl.ds(0, K)], rows.at[bn],
                                  sems.at[bn]).wait()

        @pl.when(j + 1 < nch)
        def _():
            jn = j + 1
            pltpu.async_copy(g2.at[sidx.at[(jn // GRP) % 2, jn % GRP]],
                             rows.at[bn], semg.at[bn])

        pltpu.make_async_copy(g2.at[pl.ds(0, K)], rows.at[b],
                              semg.at[b]).wait()
        pltpu.async_copy(rows.at[b], acc_sp.at[didx.at[g % 2, j % GRP]],
                         sems.at[b], add=True)
        return carry

    lax.fori_loop(0, nch, body, 0)
    pltpu.make_async_copy(g2.at[pl.ds(0, K)], rows.at[(nch - 1) % 2],
                          sems.at[(nch - 1) % 2]).wait()
    plsc.subcore_barrier()
    pltpu.sync_copy(acc_sp.at[pl.ds(base, NPT)], out.at[c, pl.ds(base, NPT)])


def _tc_scale_matmul_body(x_ref, w_ref, degs_ref, o_ref):
    deg = degs_ref[0] + degs_ref[1]
    dis = lax.rsqrt(deg)
    g = jnp.dot(x_ref[...], w_ref[...], preferred_element_type=jnp.float32)
    g = g * dis
    o_ref[0] = g[:, :D_IN]
    o_ref[1] = g[:, D_IN:]


def _tc_layer1_post_body(a_ref, degs_ref, b1_ref, w2_ref, o_ref):
    deg = degs_ref[0] + degs_ref[1]
    dis = lax.rsqrt(deg)
    h = jnp.concatenate([a_ref[0], a_ref[1]], axis=1) * dis + b1_ref[...]
    h = jnp.maximum(h, 0.0)
    g2 = jnp.dot(h, w2_ref[...], preferred_element_type=jnp.float32)
    o_ref[...] = jnp.concatenate(
        [g2 * dis, jnp.zeros((_BR, D2 - D_OUT), jnp.float32)], axis=1)


def _tc_final_body(a_ref, degs_ref, b2_ref, o_ref):
    deg = degs_ref[0] + degs_ref[1]
    dis = lax.rsqrt(deg)
    o_ref[...] = (a_ref[0, :, :D_OUT] + a_ref[1, :, :D_OUT]) * dis + b2_ref[...]


_BR = N // NB


def _deg_spec():
    return pl.BlockSpec((2, _BR, 1), lambda n: (0, n, 0))


_tc_scale_matmul = pl.pallas_call(
    _tc_scale_matmul_body,
    grid=(NB,),
    in_specs=[
        pl.BlockSpec((_BR, D_IN), lambda n: (n, 0)),
        pl.BlockSpec((D_IN, D_HID), lambda n: (0, 0)),
        _deg_spec(),
    ],
    out_specs=pl.BlockSpec((2, _BR, D_IN), lambda n: (0, n, 0)),
    out_shape=jax.ShapeDtypeStruct((2, N, D_IN), jnp.float32),
)

_tc_layer1_post = pl.pallas_call(
    _tc_layer1_post_body,
    grid=(NB,),
    in_specs=[
        pl.BlockSpec((2, _BR, D_IN), lambda n: (0, n, 0)),
        _deg_spec(),
        pl.BlockSpec((1, D_HID), lambda n: (0, 0)),
        pl.BlockSpec((D_HID, D_OUT), lambda n: (0, 0)),
    ],
    out_specs=pl.BlockSpec((_BR, D2), lambda n: (n, 0)),
    out_shape=jax.ShapeDtypeStruct((N, D2), jnp.float32),
)

_tc_final = pl.pallas_call(
    _tc_final_body,
    grid=(NB,),
    in_specs=[
        pl.BlockSpec((2, _BR, D2), lambda n: (0, n, 0)),
        _deg_spec(),
        pl.BlockSpec((1, D_OUT), lambda n: (0, 0)),
    ],
    out_specs=pl.BlockSpec((_BR, D_OUT), lambda n: (n, 0)),
    out_shape=jax.ShapeDtypeStruct((N, D_OUT), jnp.float32),
)


def kernel(data, x, edge_index, W1, b1, W2, b2):
    src = edge_index[0].astype(jnp.int32)
    dst = edge_index[1].astype(jnp.int32)
    pad = EPAD - E
    pad_i = jnp.arange(pad, dtype=jnp.int32)
    src_p = jnp.concatenate([src, pad_i % N])
    dst_p = jnp.concatenate([dst, N + pad_i % (NPAD - N)])
    src2d = src_p.reshape(CHUNKS_PAD, K)
    dst2d = dst_p.reshape(CHUNKS_PAD, K)
    srcs2 = jnp.stack([src2d, src2d + N])

    deg_init = jnp.zeros((2, NPAD, 1), jnp.float32).at[0, :N, 0].set(1.0)
    degs = _sc_degree(dst2d, deg_init)

    g1 = _tc_scale_matmul(x, W1, degs)
    agg1 = _sc_agg1(g1.reshape(2 * N, D_IN), srcs2, dst2d)

    g2 = _tc_layer1_post(agg1, degs, b1.reshape(1, D_HID), W2)
    g2init = jnp.stack([
        jnp.pad(g2, ((0, NPAD - N), (0, 0))),
        jnp.zeros((NPAD, D2), jnp.float32),
    ])
    agg2 = _sc_agg2(g2, g2init, src2d, dst2d)

    return _tc_final(agg2, degs, b2.reshape(1, D_OUT))


# speedup vs baseline: 26.1176x; 1.0049x over previous
import functools

import jax
import jax.numpy as jnp
from jax import lax
from jax.experimental import pallas as pl
from jax.experimental.pallas import tpu as pltpu
from jax.experimental.pallas import tpu_sc as plsc

N = 10000
E = 320000
D_IN = 128
D_HID = 256
D_OUT = 64
D2 = 128

NPAD = 10240
NPT = NPAD // 16
K = 128
CHUNKS_PAD = 2560
EPAD = CHUNKS_PAD * K
NB = 10
GRP = 16

_SC_MESH = plsc.VectorSubcoreMesh(core_axis_name="c", subcore_axis_name="s")


@functools.partial(
    pl.kernel,
    out_type=jax.ShapeDtypeStruct((2, NPAD, 1), jnp.float32),
    mesh=_SC_MESH,
    scratch_types=[
        pltpu.VMEM_SHARED((NPAD, 1), jnp.float32),
        pltpu.VMEM((80, K), jnp.int32),
        pltpu.VMEM((K, 1), jnp.float32),
    ],
)
def _sc_degree(dst2d, init, deg_out, deg_sp, idx_v, ones_v):
    c = lax.axis_index("c")
    s = lax.axis_index("s")
    base = s * NPT
    pltpu.sync_copy(init.at[c, pl.ds(base, NPT)], deg_sp.at[pl.ds(base, NPT)])
    pltpu.sync_copy(init.at[0, pl.ds(0, K)], ones_v)
    row0 = (c * 16 + s) * 80
    pltpu.sync_copy(dst2d.at[pl.ds(row0, 80)], idx_v)
    plsc.subcore_barrier()

    def body(j, carry):
        pltpu.sync_copy(ones_v, deg_sp.at[idx_v.at[j]], add=True)
        return carry

    lax.fori_loop(0, 80, body, 0)
    plsc.subcore_barrier()
    pltpu.sync_copy(deg_sp.at[pl.ds(base, NPT)], deg_out.at[c, pl.ds(base, NPT)])


@functools.partial(
    pl.kernel,
    out_type=jax.ShapeDtypeStruct((2, NPAD, D_IN), jnp.float32),
    mesh=_SC_MESH,
    scratch_types=[
        pltpu.VMEM_SHARED((NPAD, D_IN), jnp.float32),
        pltpu.VMEM((2, GRP, K), jnp.int32),
        pltpu.VMEM((2, GRP, K), jnp.int32),
        pltpu.VMEM((2, K, D_IN), jnp.float32),
        pltpu.VMEM((16, D_IN), jnp.float32),
        pltpu.SemaphoreType.DMA((2,)),
        pltpu.SemaphoreType.DMA((2,)),
    ],
)
def _sc_agg1(gflat, srcs2, dst2d, out, acc_sp, sidx, didx, rows, zbuf, semg,
             sems):
    c = lax.axis_index("c")
    s = lax.axis_index("s")
    base = s * 624
    pltpu.sync_copy(gflat.at[pl.ds(c * N + base, 624)],
                    acc_sp.at[pl.ds(base, 624)])

    @pl.when(s == 15)
    def _():
        pltpu.sync_copy(gflat.at[pl.ds(c * N + 9984, 16)],
                        acc_sp.at[pl.ds(9984, 16)])

    @pl.when(s == 0)
    def _():
        for i in range(16):
            for k in range(D_IN // 16):
                zbuf[i, pl.ds(k * 16, 16)] = jnp.zeros((16,), jnp.float32)
        for t in range((NPAD - N) // 16):
            pltpu.sync_copy(zbuf, acc_sp.at[pl.ds(N + t * 16, 16)])

    row0 = s * 160
    nch = 160
    pltpu.sync_copy(srcs2.at[c, pl.ds(pl.multiple_of(row0, 8), GRP)],
                    sidx.at[0])
    pltpu.sync_copy(dst2d.at[pl.ds(pl.multiple_of(row0, 8), GRP)],
                    didx.at[0])
    pltpu.async_copy(gflat.at[sidx.at[0, 0]], rows.at[0], semg.at[0])
    plsc.subcore_barrier()

    def body(j, carry):
        g = j // GRP
        b = j % 2
        bn = (j + 1) % 2

        @pl.when((j % GRP == 0) & (j + GRP < nch))
        def _():
            goff = pl.multiple_of(row0 + (g + 1) * GRP, 8)
            pltpu.sync_copy(srcs2.at[c, pl.ds(goff, GRP)],
                            sidx.at[(g + 1) % 2])
            pltpu.sync_copy(dst2d.at[pl.ds(goff, GRP)],
                            didx.at[(g + 1) % 2])

        @pl.when(j >= 1)
        def _():
            pltpu.make_async_copy(gflat.at[pl.ds(0, K)], rows.at[bn],
                                  sems.at[bn]).wait()

        @pl.when(j + 1 < nch)
        def _():
            jn = j + 1
            pltpu.async_copy(gflat.at[sidx.at[(jn // GRP) % 2, jn % GRP]],
                             rows.at[bn], semg.at[bn])

        pltpu.make_async_copy(gflat.at[pl.ds(0, K)], rows.at[b],
                              semg.at[b]).wait()
        pltpu.async_copy(rows.at[b], acc_sp.at[didx.at[g % 2, j % GRP]],
                         sems.at[b], add=True)
        return carry

    lax.fori_loop(0, nch, body, 0)
    pltpu.make_async_copy(gflat.at[pl.ds(0, K)], rows.at[(nch - 1) % 2],
                          sems.at[(nch - 1) % 2]).wait()
    plsc.subcore_barrier()
    base = s * NPT
    pltpu.sync_copy(acc_sp.at[pl.ds(base, NPT)], out.at[c, pl.ds(base, NPT)])


@functools.partial(
    pl.kernel,
    out_type=jax.ShapeDtypeStruct((2, NPAD, D2), jnp.float32),
    mesh=_SC_MESH,
    scratch_types=[
        pltpu.VMEM_SHARED((NPAD, D2), jnp.float32),
        pltpu.VMEM((2, GRP, K), jnp.int32),
        pltpu.VMEM((2, GRP, K), jnp.int32),
        pltpu.VMEM((2, K, D2), jnp.float32),
        pltpu.VMEM((16, D2), jnp.float32),
        pltpu.SemaphoreType.DMA((2,)),
        pltpu.SemaphoreType.DMA((2,)),
    ],
)
def _sc_agg2(g2, src2d, dst2d, out, acc_sp, sidx, didx, rows, zbuf, semg,
             sems):
    c = lax.axis_index("c")
    s = lax.axis_index("s")
    for i in range(16):
        for k in range(D2 // 16):
            zbuf[i, pl.ds(k * 16, 16)] = jnp.zeros((16,), jnp.float32)

    @pl.when(c == 0)
    def _():
        tb = s * 624
        pltpu.sync_copy(g2.at[pl.ds(tb, 624)], acc_sp.at[pl.ds(tb, 624)])

        @pl.when(s == 15)
        def _():
            pltpu.sync_copy(g2.at[pl.ds(9984, 16)], acc_sp.at[pl.ds(9984, 16)])

        @pl.when(s == 0)
        def _():
            for t in range((NPAD - N) // 16):
                pltpu.sync_copy(zbuf, acc_sp.at[pl.ds(N + t * 16, 16)])

    @pl.when(c == 1)
    def _():
        base = s * NPT
        for t in range(NPT // 16):
            pltpu.sync_copy(zbuf, acc_sp.at[pl.ds(base + t * 16, 16)])

    row0 = (c * 16 + s) * 80
    nch = 80
    pltpu.sync_copy(src2d.at[pl.ds(pl.multiple_of(row0, 8), GRP)], sidx.at[0])
    pltpu.sync_copy(dst2d.at[pl.ds(pl.multiple_of(row0, 8), GRP)], didx.at[0])
    pltpu.async_copy(g2.at[sidx.at[0, 0]], rows.at[0], semg.at[0])
    plsc.subcore_barrier()

    def body(j, carry):
        g = j // GRP
        b = j % 2
        bn = (j + 1) % 2

        @pl.when((j % GRP == 0) & (j + GRP < nch))
        def _():
            goff = pl.multiple_of(row0 + (g + 1) * GRP, 8)
            pltpu.sync_copy(src2d.at[pl.ds(goff, GRP)], sidx.at[(g + 1) % 2])
            pltpu.sync_copy(dst2d.at[pl.ds(goff, GRP)], didx.at[(g + 1) % 2])

        @pl.when(j >= 1)
        def _():
            pltpu.make_async_copy(g2.at[pl.ds(0, K)], rows.at[bn],
                                  sems.at[bn]).wait()

        @pl.when(j + 1 < nch)
        def _():
            jn = j + 1
            pltpu.async_copy(g2.at[sidx.at[(jn // GRP) % 2, jn % GRP]],
                             rows.at[bn], semg.at[bn])

        pltpu.make_async_copy(g2.at[pl.ds(0, K)], rows.at[b],
                              semg.at[b]).wait()
        pltpu.async_copy(rows.at[b], acc_sp.at[didx.at[g % 2, j % GRP]],
                         sems.at[b], add=True)
        return carry

    lax.fori_loop(0, nch, body, 0)
    pltpu.make_async_copy(g2.at[pl.ds(0, K)], rows.at[(nch - 1) % 2],
                          sems.at[(nch - 1) % 2]).wait()
    plsc.subcore_barrier()
    wb = s * NPT
    pltpu.sync_copy(acc_sp.at[pl.ds(wb, NPT)], out.at[c, pl.ds(wb, NPT)])


def _tc_scale_matmul_body(x_ref, w_ref, degs_ref, o_ref):
    deg = degs_ref[0] + degs_ref[1]
    dis = lax.rsqrt(deg)
    g = jnp.dot(x_ref[...], w_ref[...], preferred_element_type=jnp.float32)
    g = g * dis
    o_ref[0] = g[:, :D_IN]
    o_ref[1] = g[:, D_IN:]


def _tc_layer1_post_body(a_ref, degs_ref, b1_ref, w2_ref, o_ref):
    deg = degs_ref[0] + degs_ref[1]
    dis = lax.rsqrt(deg)
    h = jnp.concatenate([a_ref[0], a_ref[1]], axis=1) * dis + b1_ref[...]
    h = jnp.maximum(h, 0.0)
    g2 = jnp.dot(h, w2_ref[...], preferred_element_type=jnp.float32)
    o_ref[...] = jnp.concatenate(
        [g2 * dis, jnp.zeros((_BR, D2 - D_OUT), jnp.float32)], axis=1)


def _tc_final_body(a_ref, degs_ref, b2_ref, o_ref):
    deg = degs_ref[0] + degs_ref[1]
    dis = lax.rsqrt(deg)
    o_ref[...] = (a_ref[0, :, :D_OUT] + a_ref[1, :, :D_OUT]) * dis + b2_ref[...]


_BR = N // NB


def _deg_spec():
    return pl.BlockSpec((2, _BR, 1), lambda n: (0, n, 0))


_tc_scale_matmul = pl.pallas_call(
    _tc_scale_matmul_body,
    grid=(NB,),
    in_specs=[
        pl.BlockSpec((_BR, D_IN), lambda n: (n, 0)),
        pl.BlockSpec((D_IN, D_HID), lambda n: (0, 0)),
        _deg_spec(),
    ],
    out_specs=pl.BlockSpec((2, _BR, D_IN), lambda n: (0, n, 0)),
    out_shape=jax.ShapeDtypeStruct((2, N, D_IN), jnp.float32),
)

_tc_layer1_post = pl.pallas_call(
    _tc_layer1_post_body,
    grid=(NB,),
    in_specs=[
        pl.BlockSpec((2, _BR, D_IN), lambda n: (0, n, 0)),
        _deg_spec(),
        pl.BlockSpec((1, D_HID), lambda n: (0, 0)),
        pl.BlockSpec((D_HID, D_OUT), lambda n: (0, 0)),
    ],
    out_specs=pl.BlockSpec((_BR, D2), lambda n: (n, 0)),
    out_shape=jax.ShapeDtypeStruct((N, D2), jnp.float32),
)

_tc_final = pl.pallas_call(
    _tc_final_body,
    grid=(NB,),
    in_specs=[
        pl.BlockSpec((2, _BR, D2), lambda n: (0, n, 0)),
        _deg_spec(),
        pl.BlockSpec((1, D_OUT), lambda n: (0, 0)),
    ],
    out_specs=pl.BlockSpec((_BR, D_OUT), lambda n: (n, 0)),
    out_shape=jax.ShapeDtypeStruct((N, D_OUT), jnp.float32),
)


def kernel(data, x, edge_index, W1, b1, W2, b2):
    src = edge_index[0].astype(jnp.int32)
    dst = edge_index[1].astype(jnp.int32)
    pad = EPAD - E
    pad_i = jnp.arange(pad, dtype=jnp.int32)
    src_p = jnp.concatenate([src, pad_i % N])
    dst_p = jnp.concatenate([dst, N + pad_i % (NPAD - N)])
    src2d = src_p.reshape(CHUNKS_PAD, K)
    dst2d = dst_p.reshape(CHUNKS_PAD, K)
    srcs2 = jnp.stack([src2d, src2d + N])

    deg_init = jnp.zeros((2, NPAD, 1), jnp.float32).at[0, :N, 0].set(1.0)
    degs = _sc_degree(dst2d, deg_init)

    g1 = _tc_scale_matmul(x, W1, degs)
    agg1 = _sc_agg1(g1.reshape(2 * N, D_IN), srcs2, dst2d)

    g2 = _tc_layer1_post(agg1, degs, b1.reshape(1, D_HID), W2)
    agg2 = _sc_agg2(g2, src2d, dst2d)

    return _tc_final(agg2, degs, b2.reshape(1, D_OUT))
